# Initial kernel scaffold; baseline (speedup 1.0000x reference)
import jax
import jax.numpy as jnp
from jax.experimental import pallas as pl


def kernel(user_ids, item_ids, user_table, item_table, user_bias_table, item_bias_table, global_bias):
    raise NotImplementedError("write your pallas kernel here")


# speedup vs baseline: 2.0310x; 2.0310x over previous
import functools

import jax
import jax.numpy as jnp
from jax import lax
from jax.experimental import pallas as pl
from jax.experimental.pallas import tpu as pltpu
from jax.experimental.pallas import tpu_sc as plsc

N_USERS = 100000
N_ITEMS = 1000
D = 64
B = 16384

NC = 2
NS = 16
NW = NC * NS
B_PER_W = B // NW
CHUNK = 128
NCHUNK = B_PER_W // CHUNK
GROUPS = B_PER_W // 16
ROWS2D = B // CHUNK


def _sc_kernel_body(user_table, item_table, user_bias, item_bias,
                    uids2d, iids2d, x_out, b_out,
                    uidx_v, iidx_v, u_rows, v_rows, ub_v, ib_v,
                    x_v, b_v, sem):
    wid = lax.axis_index("s") * NC + lax.axis_index("c")
    row0 = wid * NCHUNK

    pltpu.sync_copy(uids2d.at[pl.ds(row0, NCHUNK)], uidx_v)
    pltpu.sync_copy(iids2d.at[pl.ds(row0, NCHUNK)], iidx_v)

    handles = []
    for c in range(NCHUNK):
        handles.append(pltpu.async_copy(
            user_table.at[uidx_v.at[c]], u_rows.at[pl.ds(c * CHUNK, CHUNK)],
            sem))
        handles.append(pltpu.async_copy(
            item_table.at[iidx_v.at[c]], v_rows.at[pl.ds(c * CHUNK, CHUNK)],
            sem))
        handles.append(pltpu.async_copy(
            user_bias.at[uidx_v.at[c]], ub_v.at[c], sem))
        handles.append(pltpu.async_copy(
            item_bias.at[iidx_v.at[c]], ib_v.at[c], sem))
    for h in handles:
        h.wait()

    lanes = lax.iota(jnp.int32, 16)
    ix8 = lax.bitwise_xor(lanes, 8)
    ix4 = lax.bitwise_xor(lanes, 4)
    ix2 = lax.bitwise_xor(lanes, 2)
    ix1 = lax.bitwise_xor(lanes, 1)
    in_lo4 = lanes < 4
    in_lo8 = lanes < 8

    def shuf(v, ix):
        return v.at[ix].get(mode="promise_in_bounds")

    def group_body(g, _):
        c = lax.shift_right_logical(g, 3)
        j = lax.bitwise_and(g, 7)
        uu = jnp.zeros((16,), jnp.float32)
        vv = jnp.zeros((16,), jnp.float32)
        dd = jnp.zeros((16,), jnp.float32)
        for jj in range(16):
            r = g * 16 + jj
            uu_p = jnp.zeros((16,), jnp.float32)
            vv_p = jnp.zeros((16,), jnp.float32)
            dd_p = jnp.zeros((16,), jnp.float32)
            for k in range(D // 16):
                uk = u_rows[r, pl.ds(k * 16, 16)]
                vk = v_rows[r, pl.ds(k * 16, 16)]
                uu_p = uu_p + uk * uk
                vv_p = vv_p + vk * vk
                dk = uk - vk
                dd_p = dd_p + dk * dk
            uu_p = uu_p + shuf(uu_p, ix8)
            vv_p = vv_p + shuf(vv_p, ix8)
            dd_p = dd_p + shuf(dd_p, ix8)
            uu_p = uu_p + shuf(uu_p, ix4)
            vv_p = vv_p + shuf(vv_p, ix4)
            dd_p = dd_p + shuf(dd_p, ix4)
            w = jnp.where(in_lo4, uu_p, jnp.where(in_lo8, vv_p, dd_p))
            w = w + shuf(w, ix2)
            w = w + shuf(w, ix1)
            m = lanes == jj
            uu = jnp.where(m, w[0], uu)
            vv = jnp.where(m, w[4], vv)
            dd = jnp.where(m, w[8], dd)
        den = (1.0 - uu) * (1.0 - vv) + 1e-6
        x = 1.0 + (2.0 * dd) / den
        ubv = ub_v[c, pl.ds(j * 16, 16)]
        ibv = ib_v[c, pl.ds(j * 16, 16)]
        bs = ubv + ibv
        x_v[c, pl.ds(j * 16, 16)] = x
        b_v[c, pl.ds(j * 16, 16)] = bs
        return _

    lax.fori_loop(0, GROUPS, group_body, None)

    pltpu.sync_copy(x_v, x_out.at[pl.ds(row0, NCHUNK)])
    pltpu.sync_copy(b_v, b_out.at[pl.ds(row0, NCHUNK)])


@jax.jit
def _sc_stage(uids2d, iids2d, user_table, item_table, user_bias, item_bias):
    mesh = plsc.VectorSubcoreMesh(core_axis_name="c", subcore_axis_name="s")
    f = functools.partial(
        pl.kernel,
        mesh=mesh,
        out_type=[
            jax.ShapeDtypeStruct((ROWS2D, CHUNK), jnp.float32),
            jax.ShapeDtypeStruct((ROWS2D, CHUNK), jnp.float32),
        ],
        scratch_types=[
            pltpu.VMEM((NCHUNK, CHUNK), jnp.int32),
            pltpu.VMEM((NCHUNK, CHUNK), jnp.int32),
            pltpu.VMEM((B_PER_W, D), jnp.float32),
            pltpu.VMEM((B_PER_W, D), jnp.float32),
            pltpu.VMEM((NCHUNK, CHUNK), jnp.float32),
            pltpu.VMEM((NCHUNK, CHUNK), jnp.float32),
            pltpu.VMEM((NCHUNK, CHUNK), jnp.float32),
            pltpu.VMEM((NCHUNK, CHUNK), jnp.float32),
            pltpu.SemaphoreType.DMA,
        ],
        compiler_params=pltpu.CompilerParams(use_tc_tiling_on_sc=False),
    )(_sc_kernel_body)
    return f(user_table, item_table, user_bias, item_bias, uids2d, iids2d)


def _tc_body(x_ref, b_ref, gb_ref, o_ref):
    x = x_ref[...]
    dist = jnp.log(x + jnp.sqrt(x * x - 1.0))
    o_ref[...] = b_ref[...] + gb_ref[0] - dist


@jax.jit
def kernel(user_ids, item_ids, user_table, item_table, user_bias_table,
           item_bias_table, global_bias):
    uids2d = user_ids.reshape(ROWS2D, CHUNK)
    iids2d = item_ids.reshape(ROWS2D, CHUNK)
    x2d, b2d = _sc_stage(uids2d, iids2d, user_table, item_table,
                         user_bias_table.reshape(-1),
                         item_bias_table.reshape(-1))
    out2d = pl.pallas_call(
        _tc_body,
        in_specs=[
            pl.BlockSpec(memory_space=pltpu.VMEM),
            pl.BlockSpec(memory_space=pltpu.VMEM),
            pl.BlockSpec(memory_space=pltpu.SMEM),
        ],
        out_shape=jax.ShapeDtypeStruct((ROWS2D, CHUNK), jnp.float32),
    )(x2d, b2d, global_bias)
    return out2d.reshape(B)
